# Initial kernel scaffold; baseline (speedup 1.0000x reference)
import jax
import jax.numpy as jnp
from jax.experimental import pallas as pl


def kernel(feat_map, boxes, w1, b1, w2, b2, fw1, fb1, fw2, fb2):
    raise NotImplementedError("write your pallas kernel here")


# speedup vs baseline: 1.0303x; 1.0303x over previous
import jax
import jax.numpy as jnp
from jax.experimental import pallas as pl
from jax.experimental.pallas import tpu as pltpu

POOL = 7
SR = 2
S = POOL * SR
PATCH = 16
C_IN = 64
C_MID = 128
FC_D = 256
NB = 64


def _kern(b_ref, r0_ref, c0_ref, fm, par_ref, w1_ref, b1_ref, w2_ref, b2_ref,
          fw1_ref, fb1_ref, fw2_ref, fb2_ref, out_ref, patch, sem):
    i = pl.program_id(0)
    copies = []
    for j in range(NB):
        r = i * NB + j
        b = b_ref[r]
        r0 = r0_ref[r]
        c0 = c0_ref[r]
        cp = pltpu.make_async_copy(
            fm.at[b, pl.ds(r0, PATCH), pl.ds(c0, PATCH), :],
            patch.at[j], sem.at[j])
        cp.start()
        copies.append(cp)
    for cp in copies:
        cp.wait()

    par = par_ref[...]
    gridv = (jax.lax.broadcasted_iota(jnp.int32, (NB, S), 1).astype(jnp.float32)
             + 0.5) / SR
    ys = par[:, 0:1] + par[:, 1:2] * gridv
    xs = par[:, 2:3] + par[:, 3:4] * gridv

    def pmat(cs):
        lo = jnp.floor(cs)
        f = (cs - lo)[:, :, None]
        loi = lo.astype(jnp.int32)[:, :, None]
        io = jax.lax.broadcasted_iota(jnp.int32, (NB, S, PATCH), 2)
        P = (jnp.where(io == loi, 1.0 - f, 0.0)
             + jnp.where(io == loi + 1, f, 0.0))
        return P.reshape(NB, POOL, SR, PATCH).sum(axis=2) * (1.0 / SR)

    Py = pmat(ys)
    Px = pmat(xs)

    pt = patch[...]
    t1 = None
    for y in range(PATCH):
        c_ = Py[:, :, y][:, :, None, None] * pt[:, y, :, :][:, None, :, :]
        t1 = c_ if t1 is None else t1 + c_
    x0 = None
    for x in range(PATCH):
        c_ = Px[:, :, x][:, :, None, None] * t1[:, :, x, :][:, None, :, :]
        x0 = c_ if x0 is None else x0 + c_

    xp = jnp.pad(x0, ((0, 0), (1, 1), (1, 1), (0, 0)))
    cols = jnp.concatenate(
        [xp[:, a:a + POOL, b_:b_ + POOL, :] for a in range(3) for b_ in range(3)],
        axis=-1).reshape(NB * POOL * POOL, 9 * C_IN)
    h1 = jnp.dot(cols, w1_ref[...], preferred_element_type=jnp.float32)
    h1 = jnp.maximum(h1 + b1_ref[...], 0.0).reshape(NB, POOL, POOL, C_MID)

    hp = jnp.pad(h1, ((0, 0), (1, 1), (1, 1), (0, 0)))
    acc = None
    t = 0
    for a in range(3):
        for b_ in range(3):
            xs_ = hp[:, a:a + POOL, b_:b_ + POOL, :].reshape(NB * POOL * POOL, C_MID)
            d = jnp.dot(xs_, w2_ref[t * C_MID:(t + 1) * C_MID, :],
                        preferred_element_type=jnp.float32)
            acc = d if acc is None else acc + d
            t += 1
    h2 = jnp.maximum(acc + b2_ref[...], 0.0).reshape(NB, POOL * POOL * C_MID)

    h3 = jnp.dot(h2, fw1_ref[...], preferred_element_type=jnp.float32)
    h3 = jnp.maximum(h3 + fb1_ref[...], 0.0)
    o = jnp.dot(h3, fw2_ref[...], preferred_element_type=jnp.float32) + fb2_ref[...]
    out_ref[...] = jax.nn.sigmoid(o[:, 0:1])


def kernel(feat_map, boxes, w1, b1, w2, b2, fw1, fb1, fw2, fb2):
    B, C, H, W = feat_map.shape
    N = boxes.shape[0]
    fm = jnp.transpose(feat_map, (0, 2, 3, 1))

    bidx = boxes[:, 0].astype(jnp.int32)
    x1 = boxes[:, 1]
    y1 = boxes[:, 2]
    bin_w = jnp.maximum(boxes[:, 3] - x1, 1.0) / POOL
    bin_h = jnp.maximum(boxes[:, 4] - y1, 1.0) / POOL
    r0 = jnp.clip(jnp.floor(y1).astype(jnp.int32), 0, H - PATCH)
    c0 = jnp.clip(jnp.floor(x1).astype(jnp.int32), 0, W - PATCH)
    z = jnp.zeros_like(x1)
    par = jnp.stack([y1 - r0.astype(jnp.float32), bin_h,
                     x1 - c0.astype(jnp.float32), bin_w, z, z, z, z], axis=1)

    w1m = jnp.transpose(w1, (3, 2, 1, 0)).reshape(9 * C_IN, C_MID)
    w2m = jnp.transpose(w2, (3, 2, 1, 0)).reshape(9 * C_MID, C_MID)
    fw1p = jnp.transpose(fw1.reshape(C_MID, POOL, POOL, FC_D),
                         (2, 1, 0, 3)).reshape(C_MID * POOL * POOL, FC_D)
    fw2p = jnp.concatenate([fw2, jnp.zeros((FC_D, 127), fw2.dtype)], axis=1)
    fb2p = jnp.concatenate([fb2, jnp.zeros((127,), fb2.dtype)]).reshape(1, 128)
    b1r = b1.reshape(1, C_MID)
    b2r = b2.reshape(1, C_MID)
    fb1r = fb1.reshape(1, FC_D)

    out = pl.pallas_call(
        _kern,
        grid_spec=pltpu.PrefetchScalarGridSpec(
            num_scalar_prefetch=3,
            grid=(N // NB,),
            in_specs=[
                pl.BlockSpec(memory_space=pl.ANY),
                pl.BlockSpec((NB, 8), lambda i, *_: (i, 0)),
                pl.BlockSpec((9 * C_IN, C_MID), lambda i, *_: (0, 0)),
                pl.BlockSpec((1, C_MID), lambda i, *_: (0, 0)),
                pl.BlockSpec((9 * C_MID, C_MID), lambda i, *_: (0, 0)),
                pl.BlockSpec((1, C_MID), lambda i, *_: (0, 0)),
                pl.BlockSpec((C_MID * POOL * POOL, FC_D), lambda i, *_: (0, 0)),
                pl.BlockSpec((1, FC_D), lambda i, *_: (0, 0)),
                pl.BlockSpec((FC_D, 128), lambda i, *_: (0, 0)),
                pl.BlockSpec((1, 128), lambda i, *_: (0, 0)),
            ],
            out_specs=pl.BlockSpec((NB, 1), lambda i, *_: (i, 0)),
            scratch_shapes=[
                pltpu.VMEM((NB, PATCH, PATCH, C_IN), jnp.float32),
                pltpu.SemaphoreType.DMA((NB,)),
            ],
        ),
        out_shape=jax.ShapeDtypeStruct((N, 1), jnp.float32),
        compiler_params=pltpu.CompilerParams(
            dimension_semantics=("parallel",)),
    )(bidx, r0, c0, fm, par, w1m, b1r, w2m, b2r, fw1p, fb1r, fw2p, fb2p)
    return out
